# baseline (speedup 1.0000x reference)
import functools

import jax
import jax.numpy as jnp
from jax import lax
from jax.experimental import pallas as pl
from jax.experimental.pallas import tpu as pltpu
from jax.experimental.pallas import tpu_sc as plsc

_N = 65536
_L = 256
_LAT = 32
_TH = 0.15
_NC, _NS, _LANES = 2, 16, 16
_NW = _NC * _NS
_ROWS_PER_W = _N // _NW
_BLK = 128
_NBLK = _ROWS_PER_W // _BLK
_NG = _BLK // _LANES


def _sc_body(x_hbm, out_hbm, x_v, o_v, sem):
    wid = lax.axis_index("s") * _NC + lax.axis_index("c")
    base = wid * _ROWS_PER_W
    iota = lax.iota(jnp.int32, _LANES)

    def blk_body(b, carry):
        row0 = base + b * _BLK
        pltpu.sync_copy(x_hbm.at[pl.ds(row0, _BLK)], x_v)

        def grp_body(g, carry2):
            rows = g * _LANES + iota

            def cond(c):
                l, p = c
                return jnp.logical_and(l < _L, jnp.any(p >= _L))

            def body(c):
                l, p = c
                col = plsc.load_gather(x_v, [rows, jnp.full((_LANES,), l, jnp.int32)])
                hit = jnp.logical_and(col > _TH, p >= _L)
                return l + 1, jnp.where(hit, l, p)

            _, p = lax.while_loop(cond, body, (0, jnp.full((_LANES,), _L, jnp.int32)))
            p = jnp.where(p >= _L, 0, p)

            fill = p.astype(jnp.float32) * (1.0 / _L)
            for j in range(_LAT):
                idx = jnp.bitwise_and(p + j, _L - 1)
                vals = plsc.load_gather(x_v, [rows, idx])
                plsc.store_scatter(o_v, [rows, jnp.full((_LANES,), j, jnp.int32)], vals)
                plsc.store_scatter(
                    o_v, [rows, jnp.full((_LANES,), _LAT + j, jnp.int32)], fill)
            return carry2

        lax.fori_loop(0, _NG, grp_body, 0)
        pltpu.sync_copy(o_v, out_hbm.at[pl.ds(row0, _BLK)])
        return carry

    lax.fori_loop(0, _NBLK, blk_body, 0)


@jax.jit
def kernel(x):
    n = x.shape[0]
    mesh = plsc.VectorSubcoreMesh(core_axis_name="c", subcore_axis_name="s")
    run = functools.partial(
        pl.kernel,
        out_type=jax.ShapeDtypeStruct((n, 2 * _LAT), jnp.float32),
        mesh=mesh,
        scratch_types=[
            pltpu.VMEM((_BLK, _L), jnp.float32),
            pltpu.VMEM((_BLK, 2 * _LAT), jnp.float32),
            pltpu.SemaphoreType.DMA,
        ],
        compiler_params=pltpu.CompilerParams(needs_layout_passes=False),
    )(_sc_body)
    out = run(x)
    return out.reshape(n, 1, 2 * _LAT)


# speedup vs baseline: 1.3195x; 1.3195x over previous
import functools

import jax
import jax.numpy as jnp
from jax import lax
from jax.experimental import pallas as pl
from jax.experimental.pallas import tpu as pltpu
from jax.experimental.pallas import tpu_sc as plsc

_N = 65536
_L = 256
_LAT = 32
_TH = 0.15
_NC, _NS, _LANES = 2, 16, 16
_NW = _NC * _NS
_ROWS_PER_W = _N // _NW
_BLK = 128
_NBLK = _ROWS_PER_W // _BLK
_NG = _BLK // _LANES
_SCAN0 = 16


def _compute_block(x_v, o_v, iota):
    def grp_body(g, rows):
        p = jnp.full((_LANES,), _L, jnp.int32)
        for l in range(_SCAN0):
            col = plsc.load_gather(x_v, [rows, jnp.full((_LANES,), l, jnp.int32)])
            p = jnp.minimum(p, jnp.where(col > _TH, l, _L))

        def tail(p0):
            def chunk(c, pc):
                base = c * _SCAN0
                for l in range(_SCAN0):
                    col = plsc.load_gather(
                        x_v, [rows, jnp.full((_LANES,), 1, jnp.int32) * (base + l)])
                    pc = jnp.minimum(pc, jnp.where(col > _TH, base + l, _L))
                return pc
            return lax.fori_loop(1, _L // _SCAN0, chunk, p0)

        p = lax.cond(jnp.any(p >= _L), tail, lambda q: q, p)
        p = jnp.where(p >= _L, 0, p)

        fill = p.astype(jnp.float32) * (1.0 / _L)
        for j in range(_LAT):
            idx = jnp.bitwise_and(p + j, _L - 1)
            vals = plsc.load_gather(x_v, [rows, idx])
            plsc.store_scatter(o_v, [rows, jnp.full((_LANES,), j, jnp.int32)], vals)
            plsc.store_scatter(
                o_v, [rows, jnp.full((_LANES,), _LAT + j, jnp.int32)], fill)
        return rows + _LANES

    lax.fori_loop(0, _NG, grp_body, iota)


def _sc_body(x_hbm, out_hbm, x0, x1, o0, o1, si0, si1, so0, so1):
    wid = lax.axis_index("s") * _NC + lax.axis_index("c")
    base = wid * _ROWS_PER_W
    iota = lax.iota(jnp.int32, _LANES)

    def in_copy(b, buf, sem):
        return pltpu.make_async_copy(
            x_hbm.at[pl.ds(base + b * _BLK, _BLK)], buf, sem)

    def out_copy(b, buf, sem):
        return pltpu.make_async_copy(
            buf, out_hbm.at[pl.ds(base + b * _BLK, _BLK)], sem)

    in_copy(0, x0, si0).start()

    def pair_body(t, carry):
        b0 = 2 * t
        in_copy(b0 + 1, x1, si1).start()
        in_copy(b0, x0, si0).wait()

        @pl.when(t > 0)
        def _():
            out_copy(2 * t - 2, o0, so0).wait()

        _compute_block(x0, o0, iota)
        out_copy(b0, o0, so0).start()

        @pl.when(t < _NBLK // 2 - 1)
        def _():
            in_copy(b0 + 2, x0, si0).start()
        in_copy(b0 + 1, x1, si1).wait()

        @pl.when(t > 0)
        def _():
            out_copy(2 * t - 1, o1, so1).wait()

        _compute_block(x1, o1, iota)
        out_copy(b0 + 1, o1, so1).start()
        return carry

    lax.fori_loop(0, _NBLK // 2, pair_body, 0)
    out_copy(_NBLK - 2, o0, so0).wait()
    out_copy(_NBLK - 1, o1, so1).wait()


@jax.jit
def kernel(x):
    n = x.shape[0]
    mesh = plsc.VectorSubcoreMesh(core_axis_name="c", subcore_axis_name="s")
    run = functools.partial(
        pl.kernel,
        out_type=jax.ShapeDtypeStruct((n, 2 * _LAT), jnp.float32),
        mesh=mesh,
        scratch_types=[
            pltpu.VMEM((_BLK, _L), jnp.float32),
            pltpu.VMEM((_BLK, _L), jnp.float32),
            pltpu.VMEM((_BLK, 2 * _LAT), jnp.float32),
            pltpu.VMEM((_BLK, 2 * _LAT), jnp.float32),
            pltpu.SemaphoreType.DMA,
            pltpu.SemaphoreType.DMA,
            pltpu.SemaphoreType.DMA,
            pltpu.SemaphoreType.DMA,
        ],
        compiler_params=pltpu.CompilerParams(needs_layout_passes=False),
    )(_sc_body)
    out = run(x)
    return out.reshape(n, 1, 2 * _LAT)


# speedup vs baseline: 1.4293x; 1.0832x over previous
import functools

import jax
import jax.numpy as jnp
from jax import lax
from jax.experimental import pallas as pl
from jax.experimental.pallas import tpu as pltpu
from jax.experimental.pallas import tpu_sc as plsc

_N = 65536
_L = 256
_LAT = 32
_TH = 0.15
_NC, _NS, _LANES = 2, 16, 16
_NW = _NC * _NS
_ROWS_PER_W = _N // _NW
_BLK = 128
_NBLK = _ROWS_PER_W // _BLK
_NG = _BLK // _LANES
_SCAN0 = 16


def _compute_block(x_v, o_v, iota):
    iota16 = lax.iota(jnp.int32, _LANES)

    def grp_body(g, rows):
        p = jnp.full((_LANES,), 16, jnp.int32)
        for k in range(_LANES):
            v = x_v[g * _LANES + k, pl.ds(0, _SCAN0)]
            f = plsc.all_reduce_ffs(v > _TH)
            p = jnp.where(iota16 == k, f, p)

        def tail(pv):
            for k in range(_LANES):
                def chunk(c, rc):
                    v = x_v[g * _LANES + k, pl.ds(c * _SCAN0, _SCAN0)]
                    f = plsc.all_reduce_ffs(v > _TH)
                    return jnp.minimum(
                        rc, jnp.where(f < 16, c * _SCAN0 + f, 1024))
                rc = lax.fori_loop(
                    1, _L // _SCAN0, chunk, jnp.full((_LANES,), 1024, jnp.int32))
                pv = jnp.where(
                    jnp.logical_and(iota16 == k, pv >= 16), rc, pv)
            return pv

        p = lax.cond(jnp.any(p >= 16), tail, lambda q: q, p)
        p = jnp.where(p >= _L, 0, p)

        fill = p.astype(jnp.float32) * (1.0 / _L)
        for j in range(_LAT):
            idx = jnp.bitwise_and(p + j, _L - 1)
            vals = plsc.load_gather(x_v, [rows, idx])
            plsc.store_scatter(o_v, [rows, jnp.full((_LANES,), j, jnp.int32)], vals)
            plsc.store_scatter(
                o_v, [rows, jnp.full((_LANES,), _LAT + j, jnp.int32)], fill)
        return rows + _LANES

    lax.fori_loop(0, _NG, grp_body, iota)


def _sc_body(x_hbm, out_hbm, x0, x1, o0, o1, si0, si1, so0, so1):
    wid = lax.axis_index("s") * _NC + lax.axis_index("c")
    base = wid * _ROWS_PER_W
    iota = lax.iota(jnp.int32, _LANES)

    def in_copy(b, buf, sem):
        return pltpu.make_async_copy(
            x_hbm.at[pl.ds(base + b * _BLK, _BLK)], buf, sem)

    def out_copy(b, buf, sem):
        return pltpu.make_async_copy(
            buf, out_hbm.at[pl.ds(base + b * _BLK, _BLK)], sem)

    in_copy(0, x0, si0).start()

    def pair_body(t, carry):
        b0 = 2 * t
        in_copy(b0 + 1, x1, si1).start()
        in_copy(b0, x0, si0).wait()

        @pl.when(t > 0)
        def _():
            out_copy(2 * t - 2, o0, so0).wait()

        _compute_block(x0, o0, iota)
        out_copy(b0, o0, so0).start()

        @pl.when(t < _NBLK // 2 - 1)
        def _():
            in_copy(b0 + 2, x0, si0).start()
        in_copy(b0 + 1, x1, si1).wait()

        @pl.when(t > 0)
        def _():
            out_copy(2 * t - 1, o1, so1).wait()

        _compute_block(x1, o1, iota)
        out_copy(b0 + 1, o1, so1).start()
        return carry

    lax.fori_loop(0, _NBLK // 2, pair_body, 0)
    out_copy(_NBLK - 2, o0, so0).wait()
    out_copy(_NBLK - 1, o1, so1).wait()


@jax.jit
def kernel(x):
    n = x.shape[0]
    mesh = plsc.VectorSubcoreMesh(core_axis_name="c", subcore_axis_name="s")
    run = functools.partial(
        pl.kernel,
        out_type=jax.ShapeDtypeStruct((n, 2 * _LAT), jnp.float32),
        mesh=mesh,
        scratch_types=[
            pltpu.VMEM((_BLK, _L), jnp.float32),
            pltpu.VMEM((_BLK, _L), jnp.float32),
            pltpu.VMEM((_BLK, 2 * _LAT), jnp.float32),
            pltpu.VMEM((_BLK, 2 * _LAT), jnp.float32),
            pltpu.SemaphoreType.DMA,
            pltpu.SemaphoreType.DMA,
            pltpu.SemaphoreType.DMA,
            pltpu.SemaphoreType.DMA,
        ],
        compiler_params=pltpu.CompilerParams(needs_layout_passes=False),
    )(_sc_body)
    out = run(x)
    return out.reshape(n, 1, 2 * _LAT)


# speedup vs baseline: 2.4848x; 1.7384x over previous
import functools

import jax
import jax.numpy as jnp
from jax import lax
from jax.experimental import pallas as pl
from jax.experimental.pallas import tpu as pltpu
from jax.experimental.pallas import tpu_sc as plsc

_N = 65536
_L = 256
_LAT = 32
_TH = 0.15
_NC, _NS, _LANES = 2, 16, 16
_NW = _NC * _NS
_ROWS_PER_W = _N // _NW
_BLK = 128
_NBLK = _ROWS_PER_W // _BLK
_NG = _BLK // _LANES
_SCAN0 = 16
_LP = _L + 1
_OP = 2 * _LAT + 1


def _compute_block(x_v, o_v, iota16):
    def grp_body(g, carry):
        rowbase = g * _LANES
        p = jnp.full((_LANES,), _SCAN0, jnp.int32)
        for k in range(_LANES):
            r = rowbase + k
            head = x_v[r, pl.ds(0, _SCAN0)]
            f = plsc.all_reduce_ffs(head > _TH)
            p = jnp.where(iota16 == k, f, p)
            rs = jnp.full((_LANES,), r, jnp.int32)
            idx1 = f + iota16
            g1 = plsc.load_gather(x_v, [rs, idx1])
            g2 = plsc.load_gather(x_v, [rs, idx1 + _SCAN0])
            fill = f.astype(jnp.float32) * (1.0 / _L)
            o_v[r, pl.ds(0, 16)] = g1
            o_v[r, pl.ds(16, 16)] = g2
            o_v[r, pl.ds(32, 16)] = fill
            o_v[r, pl.ds(48, 16)] = fill

        @pl.when(jnp.any(p >= _SCAN0))
        def _():
            pv = p
            for k in range(_LANES):
                r = rowbase + k

                def chunk(c, rc, r=r):
                    v = x_v[r, pl.ds(c * _SCAN0, _SCAN0)]
                    fc = plsc.all_reduce_ffs(v > _TH)
                    return jnp.minimum(
                        rc, jnp.where(fc < _SCAN0, c * _SCAN0 + fc, 1024))

                rc = lax.fori_loop(
                    1, _L // _SCAN0, chunk, jnp.full((_LANES,), 1024, jnp.int32))
                pv = jnp.where(
                    jnp.logical_and(iota16 == k, pv >= _SCAN0), rc, pv)
            pv = jnp.where(pv >= _L, 0, pv)
            for k in range(_LANES):
                r = rowbase + k
                ps = jnp.full((_LANES,), pv[k], jnp.int32)
                rs = jnp.full((_LANES,), r, jnp.int32)
                idx1 = jnp.bitwise_and(ps + iota16, _L - 1)
                idx2 = jnp.bitwise_and(idx1 + _SCAN0, _L - 1)
                g1 = plsc.load_gather(x_v, [rs, idx1])
                g2 = plsc.load_gather(x_v, [rs, idx2])
                fill = ps.astype(jnp.float32) * (1.0 / _L)
                o_v[r, pl.ds(0, 16)] = g1
                o_v[r, pl.ds(16, 16)] = g2
                o_v[r, pl.ds(32, 16)] = fill
                o_v[r, pl.ds(48, 16)] = fill

        return carry

    lax.fori_loop(0, _NG, grp_body, 0)


def _sc_body(x_hbm, out_hbm, x0, x1, o0, o1, si0, si1, so0, so1):
    wid = lax.axis_index("s") * _NC + lax.axis_index("c")
    base = wid * _ROWS_PER_W
    iota = lax.iota(jnp.int32, _LANES)

    def in_copy(b, buf, sem):
        return pltpu.make_async_copy(
            x_hbm.at[pl.ds(base + b * _BLK, _BLK)], buf, sem)

    def out_copy(b, buf, sem):
        return pltpu.make_async_copy(
            buf, out_hbm.at[pl.ds(base + b * _BLK, _BLK)], sem)

    in_copy(0, x0, si0).start()

    def pair_body(t, carry):
        b0 = 2 * t
        in_copy(b0 + 1, x1, si1).start()
        in_copy(b0, x0, si0).wait()

        @pl.when(t > 0)
        def _():
            out_copy(2 * t - 2, o0, so0).wait()

        _compute_block(x0, o0, iota)
        out_copy(b0, o0, so0).start()

        @pl.when(t < _NBLK // 2 - 1)
        def _():
            in_copy(b0 + 2, x0, si0).start()
        in_copy(b0 + 1, x1, si1).wait()

        @pl.when(t > 0)
        def _():
            out_copy(2 * t - 1, o1, so1).wait()

        _compute_block(x1, o1, iota)
        out_copy(b0 + 1, o1, so1).start()
        return carry

    lax.fori_loop(0, _NBLK // 2, pair_body, 0)
    out_copy(_NBLK - 2, o0, so0).wait()
    out_copy(_NBLK - 1, o1, so1).wait()


@jax.jit
def kernel(x):
    n = x.shape[0]
    mesh = plsc.VectorSubcoreMesh(core_axis_name="c", subcore_axis_name="s")
    run = functools.partial(
        pl.kernel,
        out_type=jax.ShapeDtypeStruct((n, 2 * _LAT), jnp.float32),
        mesh=mesh,
        scratch_types=[
            pltpu.VMEM((_BLK, _L), jnp.float32),
            pltpu.VMEM((_BLK, _L), jnp.float32),
            pltpu.VMEM((_BLK, 2 * _LAT), jnp.float32),
            pltpu.VMEM((_BLK, 2 * _LAT), jnp.float32),
            pltpu.SemaphoreType.DMA,
            pltpu.SemaphoreType.DMA,
            pltpu.SemaphoreType.DMA,
            pltpu.SemaphoreType.DMA,
        ],
        compiler_params=pltpu.CompilerParams(needs_layout_passes=False),
    )(_sc_body)
    out = run(x)
    return out.reshape(n, 1, 2 * _LAT)
